# baseline (speedup 1.0000x reference)
import functools

import jax
import jax.numpy as jnp
from jax import lax
from jax.experimental import pallas as pl
from jax.experimental.pallas import tpu as pltpu
from jax.experimental.pallas import tpu_sc as plsc

NC = 2
NS = 16
NW = NC * NS
LANES = 16

NUM_LAYERS = 3


def _make_deg_kernel(E, NP, KD):
    ET = E // NS
    CH = ET // KD
    SP = NP // NS
    mesh = plsc.VectorSubcoreMesh(core_axis_name="c", subcore_axis_name="s")

    @functools.partial(
        pl.kernel,
        out_type=jax.ShapeDtypeStruct((NP,), jnp.float32),
        mesh=mesh,
        scratch_types=[
            pltpu.VMEM((CH, KD), jnp.int32),
            pltpu.VMEM((KD,), jnp.float32),
            pltpu.VMEM_SHARED((NP,), jnp.float32),
        ],
    )
    def deg_kernel(col_hbm, zeros_hbm, deg_hbm, colv, onesv, acc):
        cid = lax.axis_index("c")
        sid = lax.axis_index("s")

        @pl.when(cid == 0)
        def _():
            pltpu.sync_copy(zeros_hbm.at[pl.ds(0, SP)],
                            acc.at[pl.ds(sid * SP, SP)])
            pltpu.sync_copy(col_hbm.at[sid], colv)

            def fill(i, carry):
                onesv[pl.ds(i * LANES, LANES)] = jnp.ones((16,), jnp.float32)
                return carry
            lax.fori_loop(0, KD // LANES, fill, 0)
            plsc.subcore_barrier()

            def chunk(j, carry):
                pltpu.sync_copy(onesv, acc.at[colv.at[j]], add=True)
                return carry
            lax.fori_loop(0, CH, chunk, 0)
            plsc.subcore_barrier()

            pltpu.sync_copy(acc.at[pl.ds(sid * SP, SP)],
                            deg_hbm.at[pl.ds(sid * SP, SP)])

    return deg_kernel


def _make_spmm_kernel(N, NPAD, D, E, K, SB, SUP):
    ET = E // NW
    CH = ET // K
    assert CH == SB * SUP and SB % 8 == 0
    RP = NPAD // NS
    mesh = plsc.VectorSubcoreMesh(core_axis_name="c", subcore_axis_name="s")

    @functools.partial(
        pl.kernel,
        out_type=jax.ShapeDtypeStruct((NC, NPAD, D), jnp.float32),
        mesh=mesh,
        scratch_types=[
            pltpu.VMEM((SB, K), jnp.int32),
            pltpu.VMEM((SB, K), jnp.int32),
            pltpu.VMEM((SB, K), jnp.int32),
            pltpu.VMEM((SB, K), jnp.int32),
            pltpu.VMEM((K, D), jnp.float32),
            pltpu.VMEM((K, D), jnp.float32),
            pltpu.VMEM_SHARED((NPAD, D), jnp.float32),
            pltpu.SemaphoreType.DMA,
            pltpu.SemaphoreType.DMA,
            pltpu.SemaphoreType.DMA,
            pltpu.SemaphoreType.DMA,
        ],
    )
    def spmm_kernel(g_hbm, row_hbm, col_hbm, zeros_hbm, p_hbm,
                    rowA, colA, rowB, colB, rbuf0, rbuf1, acc,
                    sem0, sem1, semr, semc):
        cid = lax.axis_index("c")
        sid = lax.axis_index("s")
        wid = sid * NC + cid

        pltpu.sync_copy(zeros_hbm, acc.at[pl.ds(sid * RP, RP)])
        pltpu.sync_copy(row_hbm.at[wid, pl.ds(0, SB)], rowA)
        pltpu.sync_copy(col_hbm.at[wid, pl.ds(0, SB)], colA)
        plsc.subcore_barrier()

        def process(rowv, colv):
            def chunk(j, carry):
                d0 = pltpu.async_copy(g_hbm.at[rowv.at[j]], rbuf0, sem0)
                d0.wait()
                pltpu.sync_copy(rbuf0, acc.at[colv.at[j]], add=True)
                return carry
            lax.fori_loop(0, SB, chunk, 0)

        for s in range(SUP):
            cur_r, cur_c = (rowA, colA) if s % 2 == 0 else (rowB, colB)
            nxt_r, nxt_c = (rowB, colB) if s % 2 == 0 else (rowA, colA)
            if s + 1 < SUP:
                dr = pltpu.async_copy(
                    row_hbm.at[wid, pl.ds((s + 1) * SB, SB)], nxt_r, semr)
                dc = pltpu.async_copy(
                    col_hbm.at[wid, pl.ds((s + 1) * SB, SB)], nxt_c, semc)
            process(cur_r, cur_c)
            if s + 1 < SUP:
                dr.wait()
                dc.wait()
        plsc.subcore_barrier()

        pltpu.sync_copy(acc.at[pl.ds(sid * RP, RP)],
                        p_hbm.at[cid, pl.ds(sid * RP, RP)])

    return spmm_kernel


def _dinv(deg):
    return jnp.where(deg > 0.5, lax.rsqrt(deg), jnp.zeros_like(deg))


def _make_scale_kernel(N, D, RB):
    grid = N // RB

    def body(x_ref, deg_ref, g_ref):
        g_ref[...] = x_ref[...] * _dinv(deg_ref[...])

    return pl.pallas_call(
        body,
        grid=(grid,),
        in_specs=[
            pl.BlockSpec((RB, D), lambda i: (i, 0)),
            pl.BlockSpec((RB, 1), lambda i: (i, 0)),
        ],
        out_specs=pl.BlockSpec((RB, D), lambda i: (i, 0)),
        out_shape=jax.ShapeDtypeStruct((N, D), jnp.float32),
    )


def _make_combine_kernel(N, D, RB, last, alpha):
    grid = N // RB

    def body_mid(p_ref, deg_ref, acc_ref, g_ref, acco_ref):
        d = _dinv(deg_ref[...])
        h = d * (p_ref[0] + p_ref[1])
        acco_ref[...] = acc_ref[...] + h
        g_ref[...] = d * h

    def body_last(p_ref, deg_ref, acc_ref, out_ref):
        d = _dinv(deg_ref[...])
        h = d * (p_ref[0] + p_ref[1])
        out_ref[...] = (acc_ref[...] + h) * alpha

    in_specs = [
        pl.BlockSpec((NC, RB, D), lambda i: (0, i, 0)),
        pl.BlockSpec((RB, 1), lambda i: (i, 0)),
        pl.BlockSpec((RB, D), lambda i: (i, 0)),
    ]
    if last:
        return pl.pallas_call(
            body_last,
            grid=(grid,),
            in_specs=in_specs,
            out_specs=pl.BlockSpec((RB, D), lambda i: (i, 0)),
            out_shape=jax.ShapeDtypeStruct((N, D), jnp.float32),
        )
    return pl.pallas_call(
        body_mid,
        grid=(grid,),
        in_specs=in_specs,
        out_specs=[pl.BlockSpec((RB, D), lambda i: (i, 0))] * 2,
        out_shape=[jax.ShapeDtypeStruct((N, D), jnp.float32)] * 2,
    )


@jax.jit
def kernel(x, edge_index):
    N, D = x.shape
    E = edge_index.shape[1]
    alpha = 1.0 / (NUM_LAYERS + 1)

    K = 80
    SB = 16
    KD = 80
    NP = ((N + NS * LANES - 1) // (NS * LANES)) * (NS * LANES)

    SUP = -(-E // (NW * K * SB))
    EP = NW * K * SB * SUP
    pad_row = jnp.zeros((EP - E,), jnp.int32)
    pad_col = jnp.full((EP - E,), N, jnp.int32)
    row = jnp.concatenate([edge_index[0], pad_row]).reshape(
        NW, EP // NW // K, K)
    col = jnp.concatenate([edge_index[1], pad_col]).reshape(
        NW, EP // NW // K, K)
    col_d = edge_index[1].reshape(NS, E // NS // KD, KD)
    zeros1 = jnp.zeros((NP // NS,), jnp.float32)
    zeros2 = jnp.zeros((NP // NS, D), jnp.float32)

    deg = _make_deg_kernel(E, NP, KD)(col_d, zeros1)
    deg_col = deg[:N, None]

    RB = 1000
    spmm = _make_spmm_kernel(N, NP, D, EP, K, SB, SUP)
    g = _make_scale_kernel(N, D, RB)(x, deg_col)
    acc = x
    out = None
    for layer in range(NUM_LAYERS):
        p = spmm(g, row, col, zeros2)
        if layer < NUM_LAYERS - 1:
            g, acc = _make_combine_kernel(N, D, RB, False, alpha)(
                p, deg_col, acc)
        else:
            out = _make_combine_kernel(N, D, RB, True, alpha)(
                p, deg_col, acc)
    return out


# speedup vs baseline: 1.1067x; 1.1067x over previous
import functools

import jax
import jax.numpy as jnp
from jax import lax
from jax.experimental import pallas as pl
from jax.experimental.pallas import tpu as pltpu
from jax.experimental.pallas import tpu_sc as plsc

NC = 2
NS = 16
NW = NC * NS
LANES = 16

NUM_LAYERS = 3


def _make_deg_kernel(E, NP, KD):
    ET = E // NS
    CH = ET // KD
    SP = NP // NS
    mesh = plsc.VectorSubcoreMesh(core_axis_name="c", subcore_axis_name="s")

    @functools.partial(
        pl.kernel,
        out_type=jax.ShapeDtypeStruct((NP,), jnp.float32),
        mesh=mesh,
        scratch_types=[
            pltpu.VMEM((CH, KD), jnp.int32),
            pltpu.VMEM((KD,), jnp.float32),
            pltpu.VMEM_SHARED((NP,), jnp.float32),
        ],
    )
    def deg_kernel(col_hbm, zeros_hbm, deg_hbm, colv, onesv, acc):
        cid = lax.axis_index("c")
        sid = lax.axis_index("s")

        @pl.when(cid == 0)
        def _():
            pltpu.sync_copy(zeros_hbm.at[pl.ds(0, SP)],
                            acc.at[pl.ds(sid * SP, SP)])
            pltpu.sync_copy(col_hbm.at[sid], colv)

            def fill(i, carry):
                onesv[pl.ds(i * LANES, LANES)] = jnp.ones((16,), jnp.float32)
                return carry
            lax.fori_loop(0, KD // LANES, fill, 0)
            plsc.subcore_barrier()

            def chunk(j, carry):
                pltpu.sync_copy(onesv, acc.at[colv.at[j]], add=True)
                return carry
            lax.fori_loop(0, CH, chunk, 0)
            plsc.subcore_barrier()

            pltpu.sync_copy(acc.at[pl.ds(sid * SP, SP)],
                            deg_hbm.at[pl.ds(sid * SP, SP)])

    return deg_kernel


def _make_spmm_kernel(N, NPAD, D, E, K):
    ET = E // NW
    CH = ET // K
    RP = NPAD // NS
    mesh = plsc.VectorSubcoreMesh(core_axis_name="c", subcore_axis_name="s")

    @functools.partial(
        pl.kernel,
        out_type=jax.ShapeDtypeStruct((NC, NPAD, D), jnp.float32),
        mesh=mesh,
        scratch_types=[
            pltpu.VMEM((CH, K), jnp.int32),
            pltpu.VMEM((CH, K), jnp.int32),
            pltpu.VMEM((K, D), jnp.float32),
            pltpu.VMEM_SHARED((NPAD, D), jnp.float32),
            pltpu.SemaphoreType.DMA,
        ],
    )
    def spmm_kernel(g_hbm, row_hbm, col_hbm, zeros_hbm, p_hbm,
                    rowv, colv, rbuf, acc, sem):
        cid = lax.axis_index("c")
        sid = lax.axis_index("s")
        wid = sid * NC + cid

        pltpu.sync_copy(zeros_hbm, acc.at[pl.ds(sid * RP, RP)])
        pltpu.sync_copy(row_hbm.at[wid], rowv)
        pltpu.sync_copy(col_hbm.at[wid], colv)
        plsc.subcore_barrier()

        def chunk(j, carry):
            pltpu.async_copy(g_hbm.at[rowv.at[j]], rbuf, sem).wait()
            pltpu.sync_copy(rbuf, acc.at[colv.at[j]], add=True)
            return carry
        lax.fori_loop(0, CH, chunk, 0)
        plsc.subcore_barrier()

        pltpu.sync_copy(acc.at[pl.ds(sid * RP, RP)],
                        p_hbm.at[cid, pl.ds(sid * RP, RP)])

    return spmm_kernel


def _dinv(deg):
    return jnp.where(deg > 0.5, lax.rsqrt(deg), jnp.zeros_like(deg))


def _make_scale_kernel(N, D, RB):
    grid = N // RB

    def body(x_ref, deg_ref, g_ref):
        g_ref[...] = x_ref[...] * _dinv(deg_ref[...])

    return pl.pallas_call(
        body,
        grid=(grid,),
        in_specs=[
            pl.BlockSpec((RB, D), lambda i: (i, 0)),
            pl.BlockSpec((RB, 1), lambda i: (i, 0)),
        ],
        out_specs=pl.BlockSpec((RB, D), lambda i: (i, 0)),
        out_shape=jax.ShapeDtypeStruct((N, D), jnp.float32),
    )


def _make_combine_kernel(N, D, RB, last, alpha):
    grid = N // RB

    def body_mid(p_ref, deg_ref, acc_ref, g_ref, acco_ref):
        d = _dinv(deg_ref[...])
        h = d * (p_ref[0] + p_ref[1])
        acco_ref[...] = acc_ref[...] + h
        g_ref[...] = d * h

    def body_last(p_ref, deg_ref, acc_ref, out_ref):
        d = _dinv(deg_ref[...])
        h = d * (p_ref[0] + p_ref[1])
        out_ref[...] = (acc_ref[...] + h) * alpha

    in_specs = [
        pl.BlockSpec((NC, RB, D), lambda i: (0, i, 0)),
        pl.BlockSpec((RB, 1), lambda i: (i, 0)),
        pl.BlockSpec((RB, D), lambda i: (i, 0)),
    ]
    if last:
        return pl.pallas_call(
            body_last,
            grid=(grid,),
            in_specs=in_specs,
            out_specs=pl.BlockSpec((RB, D), lambda i: (i, 0)),
            out_shape=jax.ShapeDtypeStruct((N, D), jnp.float32),
        )
    return pl.pallas_call(
        body_mid,
        grid=(grid,),
        in_specs=in_specs,
        out_specs=[pl.BlockSpec((RB, D), lambda i: (i, 0))] * 2,
        out_shape=[jax.ShapeDtypeStruct((N, D), jnp.float32)] * 2,
    )


@jax.jit
def kernel(x, edge_index):
    N, D = x.shape
    E = edge_index.shape[1]
    alpha = 1.0 / (NUM_LAYERS + 1)

    K = 128
    SB = 16
    KD = 80
    NP = ((N + NS * LANES - 1) // (NS * LANES)) * (NS * LANES)

    SUP = -(-E // (NW * K * SB))
    EP = NW * K * SB * SUP
    pad_row = jnp.zeros((EP - E,), jnp.int32)
    pad_col = jnp.full((EP - E,), N, jnp.int32)
    row = jnp.concatenate([edge_index[0], pad_row]).reshape(
        NW, EP // NW // K, K)
    col = jnp.concatenate([edge_index[1], pad_col]).reshape(
        NW, EP // NW // K, K)
    col_d = edge_index[1].reshape(NS, E // NS // KD, KD)
    zeros1 = jnp.zeros((NP // NS,), jnp.float32)
    zeros2 = jnp.zeros((NP // NS, D), jnp.float32)

    deg = _make_deg_kernel(E, NP, KD)(col_d, zeros1)
    deg_col = deg[:N, None]

    RB = 1000
    spmm = _make_spmm_kernel(N, NP, D, EP, K)
    g = _make_scale_kernel(N, D, RB)(x, deg_col)
    acc = x
    out = None
    for layer in range(NUM_LAYERS):
        p = spmm(g, row, col, zeros2)
        if layer < NUM_LAYERS - 1:
            g, acc = _make_combine_kernel(N, D, RB, False, alpha)(
                p, deg_col, acc)
        else:
            out = _make_combine_kernel(N, D, RB, True, alpha)(
                p, deg_col, acc)
    return out


# speedup vs baseline: 2.7491x; 2.4841x over previous
import functools

import jax
import jax.numpy as jnp
from jax import lax
from jax.experimental import pallas as pl
from jax.experimental.pallas import tpu as pltpu
from jax.experimental.pallas import tpu_sc as plsc

NC = 2
NS = 16
NW = NC * NS
LANES = 16

NUM_LAYERS = 3


def _make_deg_kernel(E, NP, KD):
    ET = E // NS
    CH = ET // KD
    SP = NP // NS
    mesh = plsc.VectorSubcoreMesh(core_axis_name="c", subcore_axis_name="s")

    @functools.partial(
        pl.kernel,
        out_type=jax.ShapeDtypeStruct((NP,), jnp.float32),
        mesh=mesh,
        scratch_types=[
            pltpu.VMEM((CH, KD), jnp.int32),
            pltpu.VMEM((KD,), jnp.float32),
            pltpu.VMEM_SHARED((NP,), jnp.float32),
        ],
    )
    def deg_kernel(col_hbm, zeros_hbm, deg_hbm, colv, onesv, acc):
        cid = lax.axis_index("c")
        sid = lax.axis_index("s")

        @pl.when(cid == 0)
        def _():
            pltpu.sync_copy(zeros_hbm.at[pl.ds(0, SP)],
                            acc.at[pl.ds(sid * SP, SP)])
            pltpu.sync_copy(col_hbm.at[sid], colv)

            def fill(i, carry):
                onesv[pl.ds(i * LANES, LANES)] = jnp.ones((16,), jnp.float32)
                return carry
            lax.fori_loop(0, KD // LANES, fill, 0)
            plsc.subcore_barrier()

            def chunk(j, carry):
                pltpu.sync_copy(onesv, acc.at[colv.at[j]], add=True)
                return carry
            lax.fori_loop(0, CH, chunk, 0)
            plsc.subcore_barrier()

            pltpu.sync_copy(acc.at[pl.ds(sid * SP, SP)],
                            deg_hbm.at[pl.ds(sid * SP, SP)])

    return deg_kernel


def _make_spmm_kernel(N, NPAD, D, E, K):
    ET = E // NW
    CH = ET // K
    RP = NPAD // NS
    mesh = plsc.VectorSubcoreMesh(core_axis_name="c", subcore_axis_name="s")

    @functools.partial(
        pl.kernel,
        out_type=jax.ShapeDtypeStruct((NC, NPAD, D), jnp.float32),
        mesh=mesh,
        scratch_types=[
            pltpu.VMEM((CH, K), jnp.int32),
            pltpu.VMEM((CH, K), jnp.int32),
            pltpu.VMEM((K, D), jnp.float32),
            pltpu.VMEM_SHARED((NPAD, D), jnp.float32),
            pltpu.SemaphoreType.DMA,
        ],
    )
    def spmm_kernel(g_hbm, row_hbm, col_hbm, zeros_hbm, p_hbm,
                    rowv, colv, rbuf, acc, sem):
        cid = lax.axis_index("c")
        sid = lax.axis_index("s")
        wid = sid * NC + cid

        pltpu.sync_copy(zeros_hbm, acc.at[pl.ds(sid * RP, RP)])
        pltpu.sync_copy(row_hbm.at[wid], rowv)
        pltpu.sync_copy(col_hbm.at[wid], colv)
        plsc.subcore_barrier()

        def chunk(j, carry):
            pltpu.async_copy(g_hbm.at[rowv.at[j]], rbuf, sem).wait()
            pltpu.sync_copy(rbuf, acc.at[colv.at[j]], add=True)
            return carry
        lax.fori_loop(0, CH, chunk, 0)
        plsc.subcore_barrier()

        pltpu.sync_copy(acc.at[pl.ds(sid * RP, RP)],
                        p_hbm.at[cid, pl.ds(sid * RP, RP)])

    return spmm_kernel


def _dinv(deg):
    return jnp.where(deg > 0.5, lax.rsqrt(deg), jnp.zeros_like(deg))


def _make_scale_kernel(N, D, RB):
    grid = N // RB

    def body(x_ref, deg_ref, g_ref):
        g_ref[...] = x_ref[...] * _dinv(deg_ref[...])

    return pl.pallas_call(
        body,
        grid=(grid,),
        in_specs=[
            pl.BlockSpec((RB, D), lambda i: (i, 0)),
            pl.BlockSpec((RB, 1), lambda i: (i, 0)),
        ],
        out_specs=pl.BlockSpec((RB, D), lambda i: (i, 0)),
        out_shape=jax.ShapeDtypeStruct((N, D), jnp.float32),
    )


def _make_combine_kernel(N, D, RB, last, alpha):
    grid = N // RB

    def body_mid(p_ref, deg_ref, acc_ref, g_ref, acco_ref):
        d = _dinv(deg_ref[...])
        h = d * (p_ref[0] + p_ref[1])
        acco_ref[...] = acc_ref[...] + h
        g_ref[...] = d * h

    def body_last(p_ref, deg_ref, acc_ref, out_ref):
        d = _dinv(deg_ref[...])
        h = d * (p_ref[0] + p_ref[1])
        out_ref[...] = (acc_ref[...] + h) * alpha

    in_specs = [
        pl.BlockSpec((NC, RB, D), lambda i: (0, i, 0)),
        pl.BlockSpec((RB, 1), lambda i: (i, 0)),
        pl.BlockSpec((RB, D), lambda i: (i, 0)),
    ]
    if last:
        return pl.pallas_call(
            body_last,
            grid=(grid,),
            in_specs=in_specs,
            out_specs=pl.BlockSpec((RB, D), lambda i: (i, 0)),
            out_shape=jax.ShapeDtypeStruct((N, D), jnp.float32),
        )
    return pl.pallas_call(
        body_mid,
        grid=(grid,),
        in_specs=in_specs,
        out_specs=[pl.BlockSpec((RB, D), lambda i: (i, 0))] * 2,
        out_shape=[jax.ShapeDtypeStruct((N, D), jnp.float32)] * 2,
    )


@jax.jit
def kernel(x, edge_index):
    N, D = x.shape
    E = edge_index.shape[1]
    alpha = 1.0 / (NUM_LAYERS + 1)

    K = 128
    SB = 16
    KD = 80
    NP = ((N + NS * LANES - 1) // (NS * LANES)) * (NS * LANES)

    SUP = -(-E // (NW * K * SB))
    EP = NW * K * SB * SUP
    pad_idx = jnp.arange(EP - E, dtype=jnp.int32)
    pad_row = pad_idx % N
    pad_col = N + pad_idx % (NP - N)
    row = jnp.concatenate([edge_index[0], pad_row]).reshape(
        NW, EP // NW // K, K)
    col = jnp.concatenate([edge_index[1], pad_col]).reshape(
        NW, EP // NW // K, K)
    col_d = edge_index[1].reshape(NS, E // NS // KD, KD)
    zeros1 = jnp.zeros((NP // NS,), jnp.float32)
    zeros2 = jnp.zeros((NP // NS, D), jnp.float32)

    deg = _make_deg_kernel(E, NP, KD)(col_d, zeros1)
    deg_col = deg[:N, None]

    RB = 1000
    spmm = _make_spmm_kernel(N, NP, D, EP, K)
    g = _make_scale_kernel(N, D, RB)(x, deg_col)
    acc = x
    out = None
    for layer in range(NUM_LAYERS):
        p = spmm(g, row, col, zeros2)
        if layer < NUM_LAYERS - 1:
            g, acc = _make_combine_kernel(N, D, RB, False, alpha)(
                p, deg_col, acc)
        else:
            out = _make_combine_kernel(N, D, RB, True, alpha)(
                p, deg_col, acc)
    return out


# speedup vs baseline: 3.1002x; 1.1277x over previous
import functools

import jax
import jax.numpy as jnp
from jax import lax
from jax.experimental import pallas as pl
from jax.experimental.pallas import tpu as pltpu
from jax.experimental.pallas import tpu_sc as plsc

NC = 2
NS = 16
NW = NC * NS
LANES = 16

NUM_LAYERS = 3


def _make_deg_kernel(E, NP, KD):
    ET = E // NS
    CH = ET // KD
    SP = NP // NS
    mesh = plsc.VectorSubcoreMesh(core_axis_name="c", subcore_axis_name="s")

    @functools.partial(
        pl.kernel,
        out_type=jax.ShapeDtypeStruct((NP,), jnp.float32),
        mesh=mesh,
        scratch_types=[
            pltpu.VMEM((CH, KD), jnp.int32),
            pltpu.VMEM((KD,), jnp.float32),
            pltpu.VMEM_SHARED((NP,), jnp.float32),
        ],
    )
    def deg_kernel(col_hbm, zeros_hbm, deg_hbm, colv, onesv, acc):
        cid = lax.axis_index("c")
        sid = lax.axis_index("s")

        @pl.when(cid == 0)
        def _():
            pltpu.sync_copy(zeros_hbm.at[pl.ds(0, SP)],
                            acc.at[pl.ds(sid * SP, SP)])
            pltpu.sync_copy(col_hbm.at[sid], colv)

            def fill(i, carry):
                onesv[pl.ds(i * LANES, LANES)] = jnp.ones((16,), jnp.float32)
                return carry
            lax.fori_loop(0, KD // LANES, fill, 0)
            plsc.subcore_barrier()

            def chunk(j, carry):
                pltpu.sync_copy(onesv, acc.at[colv.at[j]], add=True)
                return carry
            lax.fori_loop(0, CH, chunk, 0)
            plsc.subcore_barrier()

            pltpu.sync_copy(acc.at[pl.ds(sid * SP, SP)],
                            deg_hbm.at[pl.ds(sid * SP, SP)])

    return deg_kernel


def _make_spmm_kernel(N, NPAD, D, E, K):
    ET = E // NW
    CH = ET // K
    RP = NPAD // NS
    mesh = plsc.VectorSubcoreMesh(core_axis_name="c", subcore_axis_name="s")

    SB = 16
    SUP = CH // SB
    assert CH == SB * SUP

    @functools.partial(
        pl.kernel,
        out_type=jax.ShapeDtypeStruct((NC, NPAD, D), jnp.float32),
        mesh=mesh,
        scratch_types=[
            pltpu.VMEM((SB, K), jnp.int32),
            pltpu.VMEM((SB, K), jnp.int32),
            pltpu.VMEM((SB, K), jnp.int32),
            pltpu.VMEM((SB, K), jnp.int32),
            pltpu.VMEM((K, D), jnp.float32),
            pltpu.VMEM((K, D), jnp.float32),
            pltpu.VMEM_SHARED((NPAD, D), jnp.float32),
            pltpu.SemaphoreType.DMA,
            pltpu.SemaphoreType.DMA,
            pltpu.SemaphoreType.DMA,
            pltpu.SemaphoreType.DMA,
        ],
    )
    def spmm_kernel(g_hbm, row_hbm, col_hbm, zeros_hbm, p_hbm,
                    rowA, colA, rowB, colB, rbuf0, rbuf1, acc,
                    sem0, sem1, semr, semc):
        cid = lax.axis_index("c")
        sid = lax.axis_index("s")
        wid = sid * NC + cid

        pltpu.sync_copy(zeros_hbm, acc.at[pl.ds(sid * RP, RP)])
        pltpu.sync_copy(row_hbm.at[wid, pl.ds(0, SB)], rowA)
        pltpu.sync_copy(col_hbm.at[wid, pl.ds(0, SB)], colA)
        plsc.subcore_barrier()

        def process(rowv, colv):
            def chunk(j2, carry):
                j = j2 * 2
                d0 = pltpu.async_copy(g_hbm.at[rowv.at[j]], rbuf0, sem0)
                d1 = pltpu.async_copy(g_hbm.at[rowv.at[j + 1]], rbuf1,
                                      sem1)
                d0.wait()
                pltpu.sync_copy(rbuf0, acc.at[colv.at[j]], add=True)
                d1.wait()
                pltpu.sync_copy(rbuf1, acc.at[colv.at[j + 1]], add=True)
                return carry
            lax.fori_loop(0, SB // 2, chunk, 0)

        for s in range(SUP):
            cur_r, cur_c = (rowA, colA) if s % 2 == 0 else (rowB, colB)
            nxt_r, nxt_c = (rowB, colB) if s % 2 == 0 else (rowA, colA)
            if s + 1 < SUP:
                dr = pltpu.async_copy(
                    row_hbm.at[wid, pl.ds((s + 1) * SB, SB)], nxt_r, semr)
                dc = pltpu.async_copy(
                    col_hbm.at[wid, pl.ds((s + 1) * SB, SB)], nxt_c, semc)
            process(cur_r, cur_c)
            if s + 1 < SUP:
                dr.wait()
                dc.wait()
        plsc.subcore_barrier()

        pltpu.sync_copy(acc.at[pl.ds(sid * RP, RP)],
                        p_hbm.at[cid, pl.ds(sid * RP, RP)])

    return spmm_kernel


def _dinv(deg):
    return jnp.where(deg > 0.5, lax.rsqrt(deg), jnp.zeros_like(deg))


def _make_scale_kernel(N, D, RB):
    grid = N // RB

    def body(x_ref, deg_ref, g_ref):
        g_ref[...] = x_ref[...] * _dinv(deg_ref[...])

    return pl.pallas_call(
        body,
        grid=(grid,),
        in_specs=[
            pl.BlockSpec((RB, D), lambda i: (i, 0)),
            pl.BlockSpec((RB, 1), lambda i: (i, 0)),
        ],
        out_specs=pl.BlockSpec((RB, D), lambda i: (i, 0)),
        out_shape=jax.ShapeDtypeStruct((N, D), jnp.float32),
    )


def _make_combine_kernel(N, D, RB, last, alpha):
    grid = N // RB

    def body_mid(p_ref, deg_ref, acc_ref, g_ref, acco_ref):
        d = _dinv(deg_ref[...])
        h = d * (p_ref[0] + p_ref[1])
        acco_ref[...] = acc_ref[...] + h
        g_ref[...] = d * h

    def body_last(p_ref, deg_ref, acc_ref, out_ref):
        d = _dinv(deg_ref[...])
        h = d * (p_ref[0] + p_ref[1])
        out_ref[...] = (acc_ref[...] + h) * alpha

    in_specs = [
        pl.BlockSpec((NC, RB, D), lambda i: (0, i, 0)),
        pl.BlockSpec((RB, 1), lambda i: (i, 0)),
        pl.BlockSpec((RB, D), lambda i: (i, 0)),
    ]
    if last:
        return pl.pallas_call(
            body_last,
            grid=(grid,),
            in_specs=in_specs,
            out_specs=pl.BlockSpec((RB, D), lambda i: (i, 0)),
            out_shape=jax.ShapeDtypeStruct((N, D), jnp.float32),
        )
    return pl.pallas_call(
        body_mid,
        grid=(grid,),
        in_specs=in_specs,
        out_specs=[pl.BlockSpec((RB, D), lambda i: (i, 0))] * 2,
        out_shape=[jax.ShapeDtypeStruct((N, D), jnp.float32)] * 2,
    )


@jax.jit
def kernel(x, edge_index):
    N, D = x.shape
    E = edge_index.shape[1]
    alpha = 1.0 / (NUM_LAYERS + 1)

    K = 128
    SB = 16
    KD = 80
    NP = ((N + NS * LANES - 1) // (NS * LANES)) * (NS * LANES)

    SUP = -(-E // (NW * K * SB))
    EP = NW * K * SB * SUP
    pad_idx = jnp.arange(EP - E, dtype=jnp.int32)
    pad_row = pad_idx % N
    pad_col = N + pad_idx % (NP - N)
    row = jnp.concatenate([edge_index[0], pad_row]).reshape(
        NW, EP // NW // K, K)
    col = jnp.concatenate([edge_index[1], pad_col]).reshape(
        NW, EP // NW // K, K)
    col_d = edge_index[1].reshape(NS, E // NS // KD, KD)
    zeros1 = jnp.zeros((NP // NS,), jnp.float32)
    zeros2 = jnp.zeros((NP // NS, D), jnp.float32)

    deg = _make_deg_kernel(E, NP, KD)(col_d, zeros1)
    deg_col = deg[:N, None]

    RB = 1000
    spmm = _make_spmm_kernel(N, NP, D, EP, K)
    g = _make_scale_kernel(N, D, RB)(x, deg_col)
    acc = x
    out = None
    for layer in range(NUM_LAYERS):
        p = spmm(g, row, col, zeros2)
        if layer < NUM_LAYERS - 1:
            g, acc = _make_combine_kernel(N, D, RB, False, alpha)(
                p, deg_col, acc)
        else:
            out = _make_combine_kernel(N, D, RB, True, alpha)(
                p, deg_col, acc)
    return out


# speedup vs baseline: 3.1422x; 1.0135x over previous
import functools

import jax
import jax.numpy as jnp
from jax import lax
from jax.experimental import pallas as pl
from jax.experimental.pallas import tpu as pltpu
from jax.experimental.pallas import tpu_sc as plsc

NC = 2
NS = 16
NW = NC * NS
LANES = 16

NUM_LAYERS = 3


def _make_deg_kernel(E, NP, KD):
    ET = E // NS
    CH = ET // KD
    SP = NP // NS
    mesh = plsc.VectorSubcoreMesh(core_axis_name="c", subcore_axis_name="s")

    @functools.partial(
        pl.kernel,
        out_type=jax.ShapeDtypeStruct((NP,), jnp.float32),
        mesh=mesh,
        scratch_types=[
            pltpu.VMEM((CH, KD), jnp.int32),
            pltpu.VMEM((KD,), jnp.float32),
            pltpu.VMEM_SHARED((NP,), jnp.float32),
        ],
    )
    def deg_kernel(col_hbm, zeros_hbm, deg_hbm, colv, onesv, acc):
        cid = lax.axis_index("c")
        sid = lax.axis_index("s")

        @pl.when(cid == 0)
        def _():
            pltpu.sync_copy(zeros_hbm.at[pl.ds(0, SP)],
                            acc.at[pl.ds(sid * SP, SP)])
            pltpu.sync_copy(col_hbm.at[sid], colv)

            def fill(i, carry):
                onesv[pl.ds(i * LANES, LANES)] = jnp.ones((16,), jnp.float32)
                return carry
            lax.fori_loop(0, KD // LANES, fill, 0)
            plsc.subcore_barrier()

            def chunk(j, carry):
                pltpu.sync_copy(onesv, acc.at[colv.at[j]], add=True)
                return carry
            lax.fori_loop(0, CH, chunk, 0)
            plsc.subcore_barrier()

            pltpu.sync_copy(acc.at[pl.ds(sid * SP, SP)],
                            deg_hbm.at[pl.ds(sid * SP, SP)])

    return deg_kernel


def _make_spmm_kernel(N, NPAD, D, E, K):
    ET = E // NW
    CH = ET // K
    RP = NPAD // NS
    mesh = plsc.VectorSubcoreMesh(core_axis_name="c", subcore_axis_name="s")

    SB = 16
    SUP = CH // SB
    assert CH == SB * SUP

    @functools.partial(
        pl.kernel,
        out_type=jax.ShapeDtypeStruct((NC, NPAD, D), jnp.float32),
        mesh=mesh,
        scratch_types=[
            pltpu.VMEM((SB, K), jnp.int32),
            pltpu.VMEM((SB, K), jnp.int32),
            pltpu.VMEM((SB, K), jnp.int32),
            pltpu.VMEM((SB, K), jnp.int32),
            pltpu.VMEM((K, D), jnp.float32),
            pltpu.VMEM((K, D), jnp.float32),
            pltpu.VMEM_SHARED((NPAD, D), jnp.float32),
            pltpu.SemaphoreType.DMA,
            pltpu.SemaphoreType.DMA,
            pltpu.SemaphoreType.DMA,
            pltpu.SemaphoreType.DMA,
            pltpu.SemaphoreType.DMA,
            pltpu.SemaphoreType.DMA,
        ],
    )
    def spmm_kernel(g_hbm, row_hbm, col_hbm, zeros_hbm, p_hbm,
                    rowA, colA, rowB, colB, rbuf0, rbuf1, acc,
                    sem0, sem1, semr, semc, sems0, sems1):
        cid = lax.axis_index("c")
        sid = lax.axis_index("s")
        wid = sid * NC + cid

        pltpu.sync_copy(zeros_hbm, acc.at[pl.ds(sid * RP, RP)])
        pltpu.sync_copy(row_hbm.at[wid, pl.ds(0, SB)], rowA)
        pltpu.sync_copy(col_hbm.at[wid, pl.ds(0, SB)], colA)
        plsc.subcore_barrier()

        def process(rowv, colv):
            def chunk(j2, carry):
                j = j2 * 2
                d0 = pltpu.async_copy(g_hbm.at[rowv.at[j]], rbuf0, sem0)
                d1 = pltpu.async_copy(g_hbm.at[rowv.at[j + 1]], rbuf1,
                                      sem1)
                d0.wait()
                s0 = pltpu.async_copy(rbuf0, acc.at[colv.at[j]], sems0,
                                      add=True)
                d1.wait()
                s1 = pltpu.async_copy(rbuf1, acc.at[colv.at[j + 1]],
                                      sems1, add=True)
                s0.wait()
                s1.wait()
                return carry
            lax.fori_loop(0, SB // 2, chunk, 0)

        for s in range(SUP):
            cur_r, cur_c = (rowA, colA) if s % 2 == 0 else (rowB, colB)
            nxt_r, nxt_c = (rowB, colB) if s % 2 == 0 else (rowA, colA)
            if s + 1 < SUP:
                dr = pltpu.async_copy(
                    row_hbm.at[wid, pl.ds((s + 1) * SB, SB)], nxt_r, semr)
                dc = pltpu.async_copy(
                    col_hbm.at[wid, pl.ds((s + 1) * SB, SB)], nxt_c, semc)
            process(cur_r, cur_c)
            if s + 1 < SUP:
                dr.wait()
                dc.wait()
        plsc.subcore_barrier()

        pltpu.sync_copy(acc.at[pl.ds(sid * RP, RP)],
                        p_hbm.at[cid, pl.ds(sid * RP, RP)])

    return spmm_kernel


def _dinv(deg):
    return jnp.where(deg > 0.5, lax.rsqrt(deg), jnp.zeros_like(deg))


def _make_scale_kernel(N, D, RB):
    grid = N // RB

    def body(x_ref, deg_ref, g_ref):
        g_ref[...] = x_ref[...] * _dinv(deg_ref[...])

    return pl.pallas_call(
        body,
        grid=(grid,),
        in_specs=[
            pl.BlockSpec((RB, D), lambda i: (i, 0)),
            pl.BlockSpec((RB, 1), lambda i: (i, 0)),
        ],
        out_specs=pl.BlockSpec((RB, D), lambda i: (i, 0)),
        out_shape=jax.ShapeDtypeStruct((N, D), jnp.float32),
    )


def _make_combine_kernel(N, D, RB, last, alpha):
    grid = N // RB

    def body_mid(p_ref, deg_ref, acc_ref, g_ref, acco_ref):
        d = _dinv(deg_ref[...])
        h = d * (p_ref[0] + p_ref[1])
        acco_ref[...] = acc_ref[...] + h
        g_ref[...] = d * h

    def body_last(p_ref, deg_ref, acc_ref, out_ref):
        d = _dinv(deg_ref[...])
        h = d * (p_ref[0] + p_ref[1])
        out_ref[...] = (acc_ref[...] + h) * alpha

    in_specs = [
        pl.BlockSpec((NC, RB, D), lambda i: (0, i, 0)),
        pl.BlockSpec((RB, 1), lambda i: (i, 0)),
        pl.BlockSpec((RB, D), lambda i: (i, 0)),
    ]
    if last:
        return pl.pallas_call(
            body_last,
            grid=(grid,),
            in_specs=in_specs,
            out_specs=pl.BlockSpec((RB, D), lambda i: (i, 0)),
            out_shape=jax.ShapeDtypeStruct((N, D), jnp.float32),
        )
    return pl.pallas_call(
        body_mid,
        grid=(grid,),
        in_specs=in_specs,
        out_specs=[pl.BlockSpec((RB, D), lambda i: (i, 0))] * 2,
        out_shape=[jax.ShapeDtypeStruct((N, D), jnp.float32)] * 2,
    )


@jax.jit
def kernel(x, edge_index):
    N, D = x.shape
    E = edge_index.shape[1]
    alpha = 1.0 / (NUM_LAYERS + 1)

    K = 128
    SB = 16
    KD = 80
    NP = ((N + NS * LANES - 1) // (NS * LANES)) * (NS * LANES)

    SUP = -(-E // (NW * K * SB))
    EP = NW * K * SB * SUP
    pad_idx = jnp.arange(EP - E, dtype=jnp.int32)
    pad_row = pad_idx % N
    pad_col = N + pad_idx % (NP - N)
    row = jnp.concatenate([edge_index[0], pad_row]).reshape(
        NW, EP // NW // K, K)
    col = jnp.concatenate([edge_index[1], pad_col]).reshape(
        NW, EP // NW // K, K)
    col_d = edge_index[1].reshape(NS, E // NS // KD, KD)
    zeros1 = jnp.zeros((NP // NS,), jnp.float32)
    zeros2 = jnp.zeros((NP // NS, D), jnp.float32)

    deg = _make_deg_kernel(E, NP, KD)(col_d, zeros1)
    deg_col = deg[:N, None]

    RB = 1000
    spmm = _make_spmm_kernel(N, NP, D, EP, K)
    g = _make_scale_kernel(N, D, RB)(x, deg_col)
    acc = x
    out = None
    for layer in range(NUM_LAYERS):
        p = spmm(g, row, col, zeros2)
        if layer < NUM_LAYERS - 1:
            g, acc = _make_combine_kernel(N, D, RB, False, alpha)(
                p, deg_col, acc)
        else:
            out = _make_combine_kernel(N, D, RB, True, alpha)(
                p, deg_col, acc)
    return out


# speedup vs baseline: 3.1731x; 1.0099x over previous
import functools

import jax
import jax.numpy as jnp
from jax import lax
from jax.experimental import pallas as pl
from jax.experimental.pallas import tpu as pltpu
from jax.experimental.pallas import tpu_sc as plsc

NC = 2
NS = 16
NW = NC * NS
LANES = 16

NUM_LAYERS = 3


def _make_deg_kernel(E, NP, KD):
    ET = E // NW
    CH = ET // KD
    SP = NP // NS
    mesh = plsc.VectorSubcoreMesh(core_axis_name="c", subcore_axis_name="s")

    @functools.partial(
        pl.kernel,
        out_type=jax.ShapeDtypeStruct((NC, NP), jnp.float32),
        mesh=mesh,
        scratch_types=[
            pltpu.VMEM((CH, KD), jnp.int32),
            pltpu.VMEM((KD,), jnp.float32),
            pltpu.VMEM_SHARED((NP,), jnp.float32),
        ],
    )
    def deg_kernel(col_hbm, zeros_hbm, deg_hbm, colv, onesv, acc):
        cid = lax.axis_index("c")
        sid = lax.axis_index("s")
        wid = sid * NC + cid

        pltpu.sync_copy(zeros_hbm.at[pl.ds(0, SP)],
                        acc.at[pl.ds(sid * SP, SP)])
        pltpu.sync_copy(col_hbm.at[wid], colv)

        def fill(i, carry):
            onesv[pl.ds(i * LANES, LANES)] = jnp.ones((16,), jnp.float32)
            return carry
        lax.fori_loop(0, KD // LANES, fill, 0)
        plsc.subcore_barrier()

        def chunk(j, carry):
            pltpu.sync_copy(onesv, acc.at[colv.at[j]], add=True)
            return carry
        lax.fori_loop(0, CH, chunk, 0)
        plsc.subcore_barrier()

        pltpu.sync_copy(acc.at[pl.ds(sid * SP, SP)],
                        deg_hbm.at[cid, pl.ds(sid * SP, SP)])

    return deg_kernel


def _make_spmm_kernel(N, NPAD, D, E, K):
    ET = E // NW
    CH = ET // K
    RP = NPAD // NS
    mesh = plsc.VectorSubcoreMesh(core_axis_name="c", subcore_axis_name="s")

    SB = 16
    SUP = CH // SB
    assert CH == SB * SUP

    @functools.partial(
        pl.kernel,
        out_type=jax.ShapeDtypeStruct((NC, NPAD, D), jnp.float32),
        mesh=mesh,
        scratch_types=[
            pltpu.VMEM((SB, K), jnp.int32),
            pltpu.VMEM((SB, K), jnp.int32),
            pltpu.VMEM((SB, K), jnp.int32),
            pltpu.VMEM((SB, K), jnp.int32),
            pltpu.VMEM((K, D), jnp.float32),
            pltpu.VMEM((K, D), jnp.float32),
            pltpu.VMEM_SHARED((NPAD, D), jnp.float32),
            pltpu.SemaphoreType.DMA,
            pltpu.SemaphoreType.DMA,
            pltpu.SemaphoreType.DMA,
            pltpu.SemaphoreType.DMA,
            pltpu.SemaphoreType.DMA,
            pltpu.SemaphoreType.DMA,
        ],
    )
    def spmm_kernel(g_hbm, row_hbm, col_hbm, zeros_hbm, p_hbm,
                    rowA, colA, rowB, colB, rbuf0, rbuf1, acc,
                    sem0, sem1, semr, semc, sems0, sems1):
        cid = lax.axis_index("c")
        sid = lax.axis_index("s")
        wid = sid * NC + cid

        pltpu.sync_copy(zeros_hbm, acc.at[pl.ds(sid * RP, RP)])
        pltpu.sync_copy(row_hbm.at[wid, pl.ds(0, SB)], rowA)
        pltpu.sync_copy(col_hbm.at[wid, pl.ds(0, SB)], colA)
        plsc.subcore_barrier()

        def process(rowv, colv):
            def chunk(j2, carry):
                j = j2 * 2
                d0 = pltpu.async_copy(g_hbm.at[rowv.at[j]], rbuf0, sem0)
                d1 = pltpu.async_copy(g_hbm.at[rowv.at[j + 1]], rbuf1,
                                      sem1)
                d0.wait()
                s0 = pltpu.async_copy(rbuf0, acc.at[colv.at[j]], sems0,
                                      add=True)
                d1.wait()
                s1 = pltpu.async_copy(rbuf1, acc.at[colv.at[j + 1]],
                                      sems1, add=True)
                s0.wait()
                s1.wait()
                return carry
            lax.fori_loop(0, SB // 2, chunk, 0)

        for s in range(SUP):
            cur_r, cur_c = (rowA, colA) if s % 2 == 0 else (rowB, colB)
            nxt_r, nxt_c = (rowB, colB) if s % 2 == 0 else (rowA, colA)
            if s + 1 < SUP:
                dr = pltpu.async_copy(
                    row_hbm.at[wid, pl.ds((s + 1) * SB, SB)], nxt_r, semr)
                dc = pltpu.async_copy(
                    col_hbm.at[wid, pl.ds((s + 1) * SB, SB)], nxt_c, semc)
            process(cur_r, cur_c)
            if s + 1 < SUP:
                dr.wait()
                dc.wait()
        plsc.subcore_barrier()

        pltpu.sync_copy(acc.at[pl.ds(sid * RP, RP)],
                        p_hbm.at[cid, pl.ds(sid * RP, RP)])

    return spmm_kernel


def _dinv(deg):
    return jnp.where(deg > 0.5, lax.rsqrt(deg), jnp.zeros_like(deg))


def _make_scale_kernel(N, D, RB):
    grid = N // RB

    def body(x_ref, deg_ref, g_ref, dinv_ref):
        d = _dinv(deg_ref[0] + deg_ref[1])
        g_ref[...] = x_ref[...] * d
        dinv_ref[...] = d

    return pl.pallas_call(
        body,
        grid=(grid,),
        in_specs=[
            pl.BlockSpec((RB, D), lambda i: (i, 0)),
            pl.BlockSpec((NC, RB, 1), lambda i: (0, i, 0)),
        ],
        out_specs=[
            pl.BlockSpec((RB, D), lambda i: (i, 0)),
            pl.BlockSpec((RB, 1), lambda i: (i, 0)),
        ],
        out_shape=[
            jax.ShapeDtypeStruct((N, D), jnp.float32),
            jax.ShapeDtypeStruct((N, 1), jnp.float32),
        ],
    )


def _make_combine_kernel(N, D, RB, last, alpha):
    grid = N // RB

    def body_mid(p_ref, dinv_ref, acc_ref, g_ref, acco_ref):
        d = dinv_ref[...]
        h = d * (p_ref[0] + p_ref[1])
        acco_ref[...] = acc_ref[...] + h
        g_ref[...] = d * h

    def body_last(p_ref, dinv_ref, acc_ref, out_ref):
        d = dinv_ref[...]
        h = d * (p_ref[0] + p_ref[1])
        out_ref[...] = (acc_ref[...] + h) * alpha

    in_specs = [
        pl.BlockSpec((NC, RB, D), lambda i: (0, i, 0)),
        pl.BlockSpec((RB, 1), lambda i: (i, 0)),
        pl.BlockSpec((RB, D), lambda i: (i, 0)),
    ]
    if last:
        return pl.pallas_call(
            body_last,
            grid=(grid,),
            in_specs=in_specs,
            out_specs=pl.BlockSpec((RB, D), lambda i: (i, 0)),
            out_shape=jax.ShapeDtypeStruct((N, D), jnp.float32),
        )
    return pl.pallas_call(
        body_mid,
        grid=(grid,),
        in_specs=in_specs,
        out_specs=[pl.BlockSpec((RB, D), lambda i: (i, 0))] * 2,
        out_shape=[jax.ShapeDtypeStruct((N, D), jnp.float32)] * 2,
    )


@jax.jit
def kernel(x, edge_index):
    N, D = x.shape
    E = edge_index.shape[1]
    alpha = 1.0 / (NUM_LAYERS + 1)

    K = 128
    SB = 16
    KD = 80
    NP = ((N + NS * LANES - 1) // (NS * LANES)) * (NS * LANES)

    SUP = -(-E // (NW * K * SB))
    EP = NW * K * SB * SUP
    pad_idx = jnp.arange(EP - E, dtype=jnp.int32)
    pad_row = pad_idx % N
    pad_col = N + pad_idx % (NP - N)
    row = jnp.concatenate([edge_index[0], pad_row]).reshape(
        NW, EP // NW // K, K)
    col = jnp.concatenate([edge_index[1], pad_col]).reshape(
        NW, EP // NW // K, K)
    col_d = edge_index[1].reshape(NW, E // NW // KD, KD)
    zeros1 = jnp.zeros((NP // NS,), jnp.float32)
    zeros2 = jnp.zeros((NP // NS, D), jnp.float32)

    deg2 = _make_deg_kernel(E, NP, KD)(col_d, zeros1)
    deg3 = deg2[:, :, None]

    RB = 1000
    spmm = _make_spmm_kernel(N, NP, D, EP, K)
    g, dinvc = _make_scale_kernel(N, D, RB)(x, deg3)
    acc = x
    out = None
    for layer in range(NUM_LAYERS):
        p = spmm(g, row, col, zeros2)
        if layer < NUM_LAYERS - 1:
            g, acc = _make_combine_kernel(N, D, RB, False, alpha)(
                p, dinvc, acc)
        else:
            out = _make_combine_kernel(N, D, RB, True, alpha)(
                p, dinvc, acc)
    return out
